# baseline (speedup 1.0000x reference)
import jax
import jax.numpy as jnp
from jax import lax
from jax.experimental import pallas as pl
from jax.experimental.pallas import tpu as pltpu

_FMIN, _FMAX = 0.1, 0.5
_B, _T, _F = 64, 512, 1024
_ROWS = _B * _T
_TCB = 2048
_NBLK = _ROWS // _TCB
_NBUF = 4


def _tc_block_argmax(buf):
    nch = _F // 128
    best = buf[:, 0:128]
    bestj = jnp.zeros((_TCB, 128), jnp.int32)
    for j in range(1, nch):
        v = buf[:, j * 128:(j + 1) * 128]
        m = v > best
        best = jnp.maximum(best, v)
        bestj = jnp.where(m, j, bestj)
    m2 = jnp.max(best, axis=1, keepdims=True)
    lane = lax.broadcasted_iota(jnp.int32, (_TCB, 128), 1)
    key = bestj * 128 + lane
    cand = jnp.where(best == m2, key, _F)
    wini = jnp.min(cand, axis=1).astype(jnp.float32)
    scale = (_FMAX - _FMIN) / (_F - 1) * 60.0
    return wini * scale + _FMIN * 60.0


def _tc_body(x_hbm, o_hbm, bufs, out_v, sems):
    def start(i, k):
        pltpu.async_copy(x_hbm.at[pl.ds(i * _TCB, _TCB)], bufs.at[k], sems.at[k])

    def wait(i, k):
        pltpu.make_async_copy(
            x_hbm.at[pl.ds(i * _TCB, _TCB)], bufs.at[k], sems.at[k]
        ).wait()

    for k in range(_NBUF):
        start(k, k)

    def ring(p, _):
        for k in range(_NBUF):
            i = p * _NBUF + k
            wait(i, k)
            res = _tc_block_argmax(bufs.at[k])
            for r in range(_TCB // _T):
                out_v[(_TCB // _T) * i + r, :] = res[r * _T:(r + 1) * _T]

            @pl.when(i + _NBUF < _NBLK)
            def _():
                start(i + _NBUF, k)

        return 0

    lax.fori_loop(0, _NBLK // _NBUF, ring, 0)
    pltpu.sync_copy(out_v, o_hbm)


@jax.jit
def _psd_peaks(x2d):
    return pl.pallas_call(
        _tc_body,
        in_specs=[pl.BlockSpec(memory_space=pl.ANY)],
        out_specs=pl.BlockSpec(memory_space=pl.ANY),
        out_shape=jax.ShapeDtypeStruct((_B, _T), jnp.float32),
        scratch_shapes=[
            pltpu.VMEM((_NBUF, _TCB, _F), jnp.float32),
            pltpu.VMEM((_B, _T), jnp.float32),
            pltpu.SemaphoreType.DMA((_NBUF,)),
        ],
    )(x2d)


def kernel(x):
    return _psd_peaks(x.reshape(_ROWS, _F))


# speedup vs baseline: 1.0768x; 1.0768x over previous
import jax
import jax.numpy as jnp
from jax import lax
from jax.experimental import pallas as pl
from jax.experimental.pallas import tpu as pltpu

_FMIN, _FMAX = 0.1, 0.5
_B, _T, _F = 64, 512, 1024
_ROWS = _B * _T
_TCB = 1024
_NBLK = _ROWS // _TCB
_NBUF = 4


def _tc_block_argmax(buf):
    nch = _F // 128
    best = buf[:, 0:128]
    bestj = jnp.zeros((_TCB, 128), jnp.int32)
    for j in range(1, nch):
        v = buf[:, j * 128:(j + 1) * 128]
        m = v > best
        best = jnp.maximum(best, v)
        bestj = jnp.where(m, j, bestj)
    m2 = jnp.max(best, axis=1, keepdims=True)
    lane = lax.broadcasted_iota(jnp.int32, (_TCB, 128), 1)
    key = bestj * 128 + lane
    cand = jnp.where(best == m2, key, _F)
    wini = jnp.min(cand, axis=1).astype(jnp.float32)
    scale = (_FMAX - _FMIN) / (_F - 1) * 60.0
    return wini * scale + _FMIN * 60.0


def _tc_body(x_hbm, o_hbm, bufs, out_v, sems):
    def start(i, k):
        pltpu.async_copy(x_hbm.at[pl.ds(i * _TCB, _TCB)], bufs.at[k], sems.at[k])

    def wait(i, k):
        pltpu.make_async_copy(
            x_hbm.at[pl.ds(i * _TCB, _TCB)], bufs.at[k], sems.at[k]
        ).wait()

    for k in range(_NBUF):
        start(k, k)

    def ring(p, _):
        for k in range(_NBUF):
            i = p * _NBUF + k
            wait(i, k)
            res = _tc_block_argmax(bufs.at[k])
            for r in range(_TCB // _T):
                out_v[(_TCB // _T) * i + r, :] = res[r * _T:(r + 1) * _T]

            @pl.when(i + _NBUF < _NBLK)
            def _():
                start(i + _NBUF, k)

        return 0

    lax.fori_loop(0, _NBLK // _NBUF, ring, 0)
    pltpu.sync_copy(out_v, o_hbm)


@jax.jit
def _psd_peaks(x2d):
    return pl.pallas_call(
        _tc_body,
        in_specs=[pl.BlockSpec(memory_space=pl.ANY)],
        out_specs=pl.BlockSpec(memory_space=pl.ANY),
        out_shape=jax.ShapeDtypeStruct((_B, _T), jnp.float32),
        scratch_shapes=[
            pltpu.VMEM((_NBUF, _TCB, _F), jnp.float32),
            pltpu.VMEM((_B, _T), jnp.float32),
            pltpu.SemaphoreType.DMA((_NBUF,)),
        ],
    )(x2d)


def kernel(x):
    return _psd_peaks(x.reshape(_ROWS, _F))
